# baseline (speedup 1.0000x reference)
import jax
import jax.numpy as jnp
from jax import lax
from jax.experimental import pallas as pl
from jax.experimental.pallas import tpu as pltpu
from jax.experimental.pallas import tpu_sc as plsc

B = 128
N = 8192
D = 16
FLAT = N * D

NC = 2
NS = 16
L = 16
NW = NC * NS
ROWS_PER_W = B // NW

CH = 256
NCHUNK = N // CH
GROUPS = CH // L
BIG = 1 << 30


def _sc_argmax_body(x_hbm, idx_hbm, vals_hbm, buf0, buf1, valbuf, idxbuf,
                    sem0, sem1):
    w = lax.axis_index("s") * NC + lax.axis_index("c")
    iota = lax.iota(jnp.int32, L)

    def start(row, c, buf, sem):
        pltpu.make_async_copy(
            x_hbm.at[row, pl.ds(c * CH, CH)], buf, sem).start()

    def wait(buf, sem):
        pltpu.make_async_copy(x_hbm.at[0, pl.ds(0, CH)], buf, sem).wait()

    def consume(c, buf, carry):
        def group_body(g, carry):
            bv, bi = carry
            rowidx = g * L + iota
            acc = jnp.zeros((L,), jnp.float32)
            for d in range(D):
                col = jnp.full((L,), d, jnp.int32)
                v = plsc.load_gather(buf, [rowidx, col])
                acc = acc + v * v
            caps = c * CH + rowidx
            upd = acc > bv
            return jnp.where(upd, acc, bv), jnp.where(upd, caps, bi)

        return lax.fori_loop(0, GROUPS, group_body, carry)

    for t in range(ROWS_PER_W):
        row = w * ROWS_PER_W + t
        start(row, 0, buf0, sem0)

        def pair_body(j, carry, row=row):
            c0 = 2 * j
            start(row, c0 + 1, buf1, sem1)
            wait(buf0, sem0)
            carry = consume(c0, buf0, carry)

            @pl.when(j < NCHUNK // 2 - 1)
            def _():
                start(row, c0 + 2, buf0, sem0)

            wait(buf1, sem1)
            return consume(c0 + 1, buf1, carry)

        init = (jnp.full((L,), -1.0, jnp.float32), jnp.zeros((L,), jnp.int32))
        bestv, besti = lax.fori_loop(0, NCHUNK // 2, pair_body, init)

        m = jnp.max(bestv)
        idx = jnp.min(jnp.where(bestv == m, besti, jnp.int32(BIG)))
        pltpu.sync_copy(x_hbm.at[row, pl.ds(idx, 1)], valbuf)
        idxbuf[...] = jnp.full((L,), idx, jnp.int32)
        pltpu.sync_copy(idxbuf, idx_hbm.at[row])
        pltpu.sync_copy(valbuf, vals_hbm.at[pl.ds(row, 1)])


def _sc_argmax(inputs):
    mesh = plsc.VectorSubcoreMesh(core_axis_name="c", subcore_axis_name="s")
    return pl.kernel(
        _sc_argmax_body,
        out_type=[
            jax.ShapeDtypeStruct((B, L), jnp.int32),
            jax.ShapeDtypeStruct((B, D), jnp.float32),
        ],
        mesh=mesh,
        compiler_params=pltpu.CompilerParams(needs_layout_passes=False),
        scratch_types=[
            pltpu.VMEM((CH, D), jnp.float32),
            pltpu.VMEM((CH, D), jnp.float32),
            pltpu.VMEM((1, D), jnp.float32),
            pltpu.VMEM((L,), jnp.int32),
            pltpu.SemaphoreType.DMA,
            pltpu.SemaphoreType.DMA,
        ],
    )(inputs)


BB = 8


def _tc_emit_body(idx_ref, vals_ref, o_ref):
    o_ref[...] = jnp.zeros((BB, FLAT), jnp.float32)
    lane = lax.broadcasted_iota(jnp.int32, (1, 128), 1)
    for bb in range(BB):
        fmin = idx_ref[bb, 0] * D
        j0 = pl.multiple_of(fmin & ~jnp.int32(127), 128)
        l0 = fmin & jnp.int32(127)
        vrow = vals_ref[bb:bb + 1, :]
        tile = jnp.concatenate([vrow] * 8, axis=1)
        chunk = jnp.where((lane >= l0) & (lane < l0 + D), tile, 0.0)
        o_ref[bb:bb + 1, pl.ds(j0, 128)] = chunk


def kernel(inputs):
    idx2d, vals = _sc_argmax(inputs)
    out = pl.pallas_call(
        _tc_emit_body,
        grid=(B // BB,),
        in_specs=[
            pl.BlockSpec((BB, L), lambda i: (i, 0)),
            pl.BlockSpec((BB, D), lambda i: (i, 0)),
        ],
        out_specs=pl.BlockSpec((BB, FLAT), lambda i: (i, 0)),
        out_shape=jax.ShapeDtypeStruct((B, FLAT), jnp.float32),
    )(idx2d, vals)
    return out


# speedup vs baseline: 6.5120x; 6.5120x over previous
import jax
import jax.numpy as jnp
from jax import lax
from jax.experimental import pallas as pl
from jax.experimental.pallas import tpu as pltpu
from jax.experimental.pallas import tpu_sc as plsc

B = 128
N = 8192
D = 16
FLAT = N * D

NC = 2
NS = 16
L = 16
NW = NC * NS
ROWS_PER_W = B // NW

W = 2048
NCHUNK = N // W
GROUPS = W // L
BIG = 1 << 30


def _sc_argmax_body(x_hbm, idx_hbm, vals_hbm, buf0, buf1, valbuf, outv,
                    idxbuf, sem0, sem1):
    w = lax.axis_index("s") * NC + lax.axis_index("c")
    iota = lax.iota(jnp.int32, L)

    def start(row, c, buf, sem):
        pltpu.make_async_copy(
            x_hbm.at[row, :, pl.ds(c * W, W)], buf, sem).start()

    def wait(buf, sem):
        pltpu.make_async_copy(x_hbm.at[0, :, pl.ds(0, W)], buf, sem).wait()

    def consume(c, buf, carry):
        def group_body(g, carry):
            bv, bi = carry
            col = g * L + iota
            acc = jnp.zeros((L,), jnp.float32)
            for d in range(D):
                v = plsc.load_gather(buf, [jnp.full((L,), d, jnp.int32), col])
                acc = acc + v * v
            caps = c * W + col
            upd = acc > bv
            return jnp.where(upd, acc, bv), jnp.where(upd, caps, bi)

        return lax.fori_loop(0, GROUPS, group_body, carry)

    for t in range(ROWS_PER_W):
        row = w * ROWS_PER_W + t
        start(row, 0, buf0, sem0)

        def pair_body(j, carry, row=row):
            c0 = 2 * j
            start(row, c0 + 1, buf1, sem1)
            wait(buf0, sem0)
            carry = consume(c0, buf0, carry)

            @pl.when(j < NCHUNK // 2 - 1)
            def _():
                start(row, c0 + 2, buf0, sem0)

            wait(buf1, sem1)
            return consume(c0 + 1, buf1, carry)

        init = (jnp.full((L,), -1.0, jnp.float32), jnp.zeros((L,), jnp.int32))
        bestv, besti = lax.fori_loop(0, NCHUNK // 2, pair_body, init)

        m = jnp.max(bestv)
        idx = jnp.min(jnp.where(bestv == m, besti, jnp.int32(BIG)))
        tile0 = pl.multiple_of(idx & ~jnp.int32(127), 128)
        pltpu.sync_copy(x_hbm.at[row, :, pl.ds(tile0, 128)], valbuf)
        outv[...] = plsc.load_gather(
            valbuf, [iota, jnp.full((L,), idx & 127, jnp.int32)])
        idxbuf[...] = jnp.full((L,), idx, jnp.int32)
        pltpu.sync_copy(idxbuf, idx_hbm.at[row])
        pltpu.sync_copy(outv, vals_hbm.at[row])


def _sc_argmax(x_t):
    mesh = plsc.VectorSubcoreMesh(core_axis_name="c", subcore_axis_name="s")
    return pl.kernel(
        _sc_argmax_body,
        out_type=[
            jax.ShapeDtypeStruct((B, L), jnp.int32),
            jax.ShapeDtypeStruct((B, D), jnp.float32),
        ],
        mesh=mesh,
        compiler_params=pltpu.CompilerParams(needs_layout_passes=False),
        scratch_types=[
            pltpu.VMEM((D, W), jnp.float32),
            pltpu.VMEM((D, W), jnp.float32),
            pltpu.VMEM((D, 128), jnp.float32),
            pltpu.VMEM((L,), jnp.float32),
            pltpu.VMEM((L,), jnp.int32),
            pltpu.SemaphoreType.DMA,
            pltpu.SemaphoreType.DMA,
        ],
    )(x_t)


BB = 8


def _tc_emit_body(idx_ref, vals_ref, o_ref):
    o_ref[...] = jnp.zeros((BB, FLAT), jnp.float32)
    lane = lax.broadcasted_iota(jnp.int32, (1, 128), 1)
    for bb in range(BB):
        fmin = idx_ref[bb, 0] * D
        j0 = pl.multiple_of(fmin & ~jnp.int32(127), 128)
        l0 = fmin & jnp.int32(127)
        vrow = vals_ref[bb:bb + 1, :]
        tile = jnp.concatenate([vrow] * 8, axis=1)
        chunk = jnp.where((lane >= l0) & (lane < l0 + D), tile, 0.0)
        o_ref[bb:bb + 1, pl.ds(j0, 128)] = chunk


def kernel(inputs):
    x_t = jnp.transpose(inputs, (0, 2, 1))
    idx2d, vals = _sc_argmax(x_t)
    out = pl.pallas_call(
        _tc_emit_body,
        grid=(B // BB,),
        in_specs=[
            pl.BlockSpec((BB, L), lambda i: (i, 0)),
            pl.BlockSpec((BB, D), lambda i: (i, 0)),
        ],
        out_specs=pl.BlockSpec((BB, FLAT), lambda i: (i, 0)),
        out_shape=jax.ShapeDtypeStruct((B, FLAT), jnp.float32),
    )(idx2d, vals)
    return out


# speedup vs baseline: 7.3750x; 1.1325x over previous
import jax
import jax.numpy as jnp
from jax import lax
from jax.experimental import pallas as pl
from jax.experimental.pallas import tpu as pltpu
from jax.experimental.pallas import tpu_sc as plsc

B = 128
N = 8192
D = 16
FLAT = N * D

NC = 2
NS = 16
L = 16
NW = NC * NS
ROWS_PER_W = B // NW

W = 2048
NCHUNK = N // W
GROUPS = W // L
BIG = 1 << 30


def _sc_argmax_body(x_hbm, idx_hbm, vals_hbm, buf0, buf1, valbuf, outv,
                    idxbuf, sem0, sem1):
    w = lax.axis_index("s") * NC + lax.axis_index("c")
    iota = lax.iota(jnp.int32, L)

    def start(row, c, buf, sem):
        pltpu.make_async_copy(
            x_hbm.at[row, :, pl.ds(c * W, W)], buf, sem).start()

    def wait(buf, sem):
        pltpu.make_async_copy(x_hbm.at[0, :, pl.ds(0, W)], buf, sem).wait()

    def consume(c, buf, carry):
        def group_body(g, carry):
            bv, bi = carry
            col = g * L + iota
            acc = jnp.zeros((L,), jnp.float32)
            for d in range(D):
                v = plsc.load_gather(buf, [jnp.full((L,), d, jnp.int32), col])
                acc = acc + v * v
            caps = c * W + col
            upd = acc > bv
            return jnp.where(upd, acc, bv), jnp.where(upd, caps, bi)

        return lax.fori_loop(0, GROUPS, group_body, carry)

    for t in range(ROWS_PER_W):
        row = w * ROWS_PER_W + t
        start(row, 0, buf0, sem0)

        def pair_body(j, carry, row=row):
            c0 = 2 * j
            start(row, c0 + 1, buf1, sem1)
            wait(buf0, sem0)
            carry = consume(c0, buf0, carry)

            @pl.when(j < NCHUNK // 2 - 1)
            def _():
                start(row, c0 + 2, buf0, sem0)

            wait(buf1, sem1)
            return consume(c0 + 1, buf1, carry)

        init = (jnp.full((L,), -1.0, jnp.float32), jnp.zeros((L,), jnp.int32))
        bestv, besti = lax.fori_loop(0, NCHUNK // 2, pair_body, init)

        m = jnp.max(bestv)
        idx = jnp.min(jnp.where(bestv == m, besti, jnp.int32(BIG)))
        tile0 = pl.multiple_of(idx & ~jnp.int32(127), 128)
        pltpu.sync_copy(x_hbm.at[row, :, pl.ds(tile0, 128)], valbuf)
        outv[...] = plsc.load_gather(
            valbuf, [iota, jnp.full((L,), idx & 127, jnp.int32)])
        idxbuf[...] = jnp.full((L,), idx, jnp.int32)
        pltpu.sync_copy(idxbuf, idx_hbm.at[row])
        pltpu.sync_copy(outv, vals_hbm.at[row])


def _sc_argmax(x_t):
    mesh = plsc.VectorSubcoreMesh(core_axis_name="c", subcore_axis_name="s")
    return pl.kernel(
        _sc_argmax_body,
        out_type=[
            jax.ShapeDtypeStruct((B, L), jnp.int32),
            jax.ShapeDtypeStruct((B, D), jnp.float32),
        ],
        mesh=mesh,
        compiler_params=pltpu.CompilerParams(needs_layout_passes=False),
        scratch_types=[
            pltpu.VMEM((D, W), jnp.float32),
            pltpu.VMEM((D, W), jnp.float32),
            pltpu.VMEM((D, 128), jnp.float32),
            pltpu.VMEM((L,), jnp.float32),
            pltpu.VMEM((L,), jnp.int32),
            pltpu.SemaphoreType.DMA,
            pltpu.SemaphoreType.DMA,
        ],
    )(x_t)


BB = 8


def _tc_zeros_body(o_ref):
    o_ref[...] = jnp.zeros((BB, FLAT), jnp.float32)


def _tc_insert_body(z_hbm, idx_ref, vals_ref, o_hbm, chunks, sem):
    del z_hbm
    lane = lax.broadcasted_iota(jnp.int32, (1, 128), 1)
    l0 = (idx_ref[...][:, 0:1] * D) & 127
    tile = jnp.concatenate([vals_ref[...]] * 8, axis=1)
    chunks[...] = jnp.where((lane >= l0) & (lane < l0 + D), tile, 0.0)
    for b in range(B):
        fmin = idx_ref[b, 0] * D
        j0 = pl.multiple_of(fmin & ~jnp.int32(127), 128)
        pltpu.make_async_copy(
            chunks.at[b, pl.ds(0, 128)], o_hbm.at[b, pl.ds(j0, 128)],
            sem).start()
    for b in range(B):
        pltpu.make_async_copy(
            chunks.at[0, pl.ds(0, 128)], o_hbm.at[0, pl.ds(0, 128)],
            sem).wait()


def kernel(inputs):
    x_t = jnp.transpose(inputs, (0, 2, 1))
    idx2d, vals = _sc_argmax(x_t)
    zeros = pl.pallas_call(
        _tc_zeros_body,
        grid=(B // BB,),
        out_specs=pl.BlockSpec((BB, FLAT), lambda i: (i, 0)),
        out_shape=jax.ShapeDtypeStruct((B, FLAT), jnp.float32),
    )()
    out = pl.pallas_call(
        _tc_insert_body,
        in_specs=[
            pl.BlockSpec(memory_space=pltpu.HBM),
            pl.BlockSpec((B, L), lambda: (0, 0)),
            pl.BlockSpec((B, D), lambda: (0, 0)),
        ],
        out_specs=pl.BlockSpec(memory_space=pltpu.HBM),
        out_shape=jax.ShapeDtypeStruct((B, FLAT), jnp.float32),
        input_output_aliases={0: 0},
        scratch_shapes=[
            pltpu.VMEM((B, 128), jnp.float32),
            pltpu.SemaphoreType.DMA,
        ],
    )(zeros, idx2d, vals)
    return out
